# Initial kernel scaffold; baseline (speedup 1.0000x reference)
import jax
import jax.numpy as jnp
from jax.experimental import pallas as pl


def kernel(x, table, W_num, b_num):
    raise NotImplementedError("write your pallas kernel here")


# speedup vs baseline: 2.0785x; 2.0785x over previous
import functools

import jax
import jax.numpy as jnp
from jax import lax
from jax.experimental import pallas as pl
from jax.experimental.pallas import tpu as pltpu
from jax.experimental.pallas import tpu_sc as plsc

NUM_CAT = 26
N_FIELDS = 39
N_NUM = N_FIELDS - NUM_CAT
DIM = 128
BATCH = 16384

NC, NS = 2, 16
NW = NC * NS
BT = BATCH * NUM_CAT
PW = BT // NW
CH = 128
NCH = PW // CH

_sc_mesh = plsc.VectorSubcoreMesh(core_axis_name="c", subcore_axis_name="s")


@functools.partial(
    pl.kernel,
    out_type=jax.ShapeDtypeStruct((BT, DIM), jnp.float32),
    mesh=_sc_mesh,
    scratch_types=[
        pltpu.VMEM((NCH, CH), jnp.int32),
        pltpu.VMEM((CH, DIM), jnp.float32),
        pltpu.SemaphoreType.DMA,
    ],
)
def _sc_gather(table_hbm, idx_hbm, out_hbm, idx_v, rows_v, sem):
    wid = lax.axis_index("s") * NC + lax.axis_index("c")
    pltpu.sync_copy(idx_hbm.at[wid], idx_v)
    base = wid * PW

    def chunk(c, carry):
        pltpu.async_copy(table_hbm.at[idx_v.at[c]], rows_v, sem).wait()
        pltpu.sync_copy(rows_v, out_hbm.at[pl.ds(base + c * CH, CH)])
        return carry

    lax.fori_loop(0, NCH, chunk, 0)


def _combine_body(cat_ref, num_ref, w_ref, b_ref, out_ref):
    w = w_ref[0, :]
    b = b_ref[0, :]
    dense = jnp.maximum(
        num_ref[...][:, :, None] * w[None, None, :] + b[None, None, :], 0.0
    )
    out_ref[...] = jnp.concatenate([cat_ref[...], dense], axis=1)


def _combine(cat3, num, w, b):
    BB = 256
    grid = BATCH // BB
    return pl.pallas_call(
        _combine_body,
        grid=(grid,),
        in_specs=[
            pl.BlockSpec((BB, NUM_CAT, DIM), lambda i: (i, 0, 0)),
            pl.BlockSpec((BB, N_NUM), lambda i: (i, 0)),
            pl.BlockSpec((1, DIM), lambda i: (0, 0)),
            pl.BlockSpec((1, DIM), lambda i: (0, 0)),
        ],
        out_specs=pl.BlockSpec((BB, N_FIELDS, DIM), lambda i: (i, 0, 0)),
        out_shape=jax.ShapeDtypeStruct((BATCH, N_FIELDS, DIM), jnp.float32),
    )(cat3, num, w, b)


def kernel(x, table, W_num, b_num):
    idx = x[:, :NUM_CAT].astype(jnp.int32).reshape(NW, NCH, CH)
    num = x[:, NUM_CAT:]
    cat = _sc_gather(table, idx)
    cat3 = cat.reshape(BATCH, NUM_CAT, DIM)
    return _combine(cat3, num, W_num.reshape(1, DIM), b_num.reshape(1, DIM))
